# baseline (speedup 1.0000x reference)
import functools

import jax
import jax.numpy as jnp
from jax import lax
from jax.experimental import pallas as pl
from jax.experimental.pallas import tpu as pltpu
from jax.experimental.pallas import tpu_sc as plsc

_B = 4096
_L = 200
_DIM = 64
_LP = 104
_NW = 32
_BPW = _B // _NW
_G = 2
_NG = _BPW // _G
_GR = 2 * _LP * _G


def _make_kernel():
    mesh = plsc.VectorSubcoreMesh(core_axis_name="c", subcore_axis_name="s")

    @functools.partial(
        pl.kernel,
        mesh=mesh,
        out_type=jax.ShapeDtypeStruct((_B, _DIM), jnp.float32),
        compiler_params=pltpu.CompilerParams(use_tc_tiling_on_sc=False),
        scratch_types=[
            pltpu.VMEM((_BPW, 2, _LP), jnp.int32),
            pltpu.VMEM((2, _GR, _DIM), jnp.float32),
            pltpu.VMEM((_BPW, _DIM), jnp.float32),
            pltpu.SemaphoreType.DMA,
            pltpu.SemaphoreType.DMA,
        ],
    )
    def k(x_hbm, table_hbm, out_hbm, idx_v, rows_v, out_v, sem_a, sem_b):
        wid = lax.axis_index("s") * 2 + lax.axis_index("c")
        base = wid * _BPW
        pltpu.sync_copy(x_hbm.at[pl.ds(base, _BPW)], idx_v)

        def issue(g, buf, sem):
            for j in range(_G):
                for h in range(2):
                    pltpu.async_copy(
                        table_hbm.at[idx_v.at[g * _G + j, h]],
                        rows_v.at[buf, pl.ds((2 * j + h) * _LP, _LP)],
                        sem)

        def drain(buf, sem):
            pltpu.make_async_copy(
                out_hbm.at[pl.ds(0, _GR)], rows_v.at[buf], sem).wait()

        def accumulate(g, buf):
            for j in range(_G):
                cb = 2 * j * _LP

                def acc_body(r, accs):
                    return tuple(
                        a
                        + rows_v[buf, cb + r, pl.ds(16 * c, 16)]
                        + rows_v[buf, cb + _LP + r, pl.ds(16 * c, 16)]
                        for c, a in enumerate(accs)
                    )

                accs = lax.fori_loop(
                    0, _LP, acc_body,
                    tuple(jnp.zeros((16,), jnp.float32) for _ in range(4)))
                for c in range(4):
                    out_v[g * _G + j, pl.ds(16 * c, 16)] = accs[c] * (1.0 / _L)

        issue(0, 0, sem_a)

        def step(s, carry):
            g0 = 2 * s
            issue(g0 + 1, 1, sem_b)
            drain(0, sem_a)
            accumulate(g0, 0)

            @pl.when(s < _NG // 2 - 1)
            def _():
                issue(g0 + 2, 0, sem_a)

            drain(1, sem_b)
            accumulate(g0 + 1, 1)
            return carry

        lax.fori_loop(0, _NG // 2, step, 0)
        pltpu.sync_copy(out_v, out_hbm.at[pl.ds(base, _BPW)])

    return k


_kernel_call = _make_kernel()


def kernel(X, table):
    Xp = jnp.pad(X.reshape(_B, 2, _L // 2), ((0, 0), (0, 0), (0, _LP - _L // 2)))
    return _kernel_call(Xp, table)


# speedup vs baseline: 1.0003x; 1.0003x over previous
import functools

import jax
import jax.numpy as jnp
from jax import lax
from jax.experimental import pallas as pl
from jax.experimental.pallas import tpu as pltpu
from jax.experimental.pallas import tpu_sc as plsc

_B = 4096
_L = 200
_DIM = 64
_LP = 104
_NW = 32
_BPW = _B // _NW
_G = 2
_NG = _BPW // _G
_GR = 2 * _LP * _G


def _make_kernel():
    mesh = plsc.VectorSubcoreMesh(core_axis_name="c", subcore_axis_name="s")

    @functools.partial(
        pl.kernel,
        mesh=mesh,
        out_type=jax.ShapeDtypeStruct((_B, _DIM), jnp.float32),
        compiler_params=pltpu.CompilerParams(use_tc_tiling_on_sc=False),
        scratch_types=[
            pltpu.VMEM((_BPW, 2, _LP), jnp.int32),
            pltpu.VMEM((2, _GR, _DIM), jnp.float32),
            pltpu.VMEM((_BPW, _DIM), jnp.float32),
            pltpu.SemaphoreType.DMA,
            pltpu.SemaphoreType.DMA,
        ],
    )
    def k(x_hbm, table_hbm, out_hbm, idx_v, rows_v, out_v, sem_a, sem_b):
        wid = lax.axis_index("s") * 2 + lax.axis_index("c")
        base = wid * _BPW
        pltpu.sync_copy(x_hbm.at[pl.ds(base, _BPW)], idx_v)

        def issue(g, buf, sem):
            for j in range(_G):
                for h in range(2):
                    pltpu.async_copy(
                        table_hbm.at[idx_v.at[g * _G + j, h]],
                        rows_v.at[buf, pl.ds((2 * j + h) * _LP, _LP)],
                        sem)

        def drain(buf, sem):
            pltpu.make_async_copy(
                out_hbm.at[pl.ds(0, _GR)], rows_v.at[buf], sem).wait()

        def accumulate(g, buf):
            for j in range(_G):
                cb = 2 * j * _LP

                def acc_body(q, accs):
                    accs = list(accs)
                    base = cb + q * 16
                    for rr in range(16):
                        gidx = rr % 4
                        for c in range(4):
                            accs[4 * gidx + c] = (
                                accs[4 * gidx + c]
                                + rows_v[buf, base + rr, pl.ds(16 * c, 16)])
                    return tuple(accs)

                accs = lax.fori_loop(
                    0, 2 * _LP // 16, acc_body,
                    tuple(jnp.zeros((16,), jnp.float32) for _ in range(16)))
                for c in range(4):
                    tot = ((accs[c] + accs[4 + c])
                           + (accs[8 + c] + accs[12 + c]))
                    out_v[g * _G + j, pl.ds(16 * c, 16)] = tot * (1.0 / _L)

        issue(0, 0, sem_a)

        def step(s, carry):
            g0 = 2 * s
            issue(g0 + 1, 1, sem_b)
            drain(0, sem_a)
            accumulate(g0, 0)

            @pl.when(s < _NG // 2 - 1)
            def _():
                issue(g0 + 2, 0, sem_a)

            drain(1, sem_b)
            accumulate(g0 + 1, 1)
            return carry

        lax.fori_loop(0, _NG // 2, step, 0)
        pltpu.sync_copy(out_v, out_hbm.at[pl.ds(base, _BPW)])

    return k


_kernel_call = _make_kernel()


def kernel(X, table):
    Xp = jnp.pad(X.reshape(_B, 2, _L // 2), ((0, 0), (0, 0), (0, _LP - _L // 2)))
    return _kernel_call(Xp, table)


# speedup vs baseline: 1.8681x; 1.8676x over previous
import functools

import jax
import jax.numpy as jnp
from jax import lax
from jax.experimental import pallas as pl
from jax.experimental.pallas import tpu as pltpu
from jax.experimental.pallas import tpu_sc as plsc

_B = 4096
_L = 200
_DIM = 64
_LP = 104
_NW = 32
_BPW = _B // _NW
_G = 2
_NG = _BPW // _G
_GR = 2 * _LP * _G


def _make_kernel():
    mesh = plsc.VectorSubcoreMesh(core_axis_name="c", subcore_axis_name="s")

    @functools.partial(
        pl.kernel,
        mesh=mesh,
        out_type=jax.ShapeDtypeStruct((_B, _DIM), jnp.float32),
        compiler_params=pltpu.CompilerParams(use_tc_tiling_on_sc=False),
        scratch_types=[
            pltpu.VMEM((_BPW, 2, _LP), jnp.int32),
            pltpu.VMEM((2, _GR, _DIM), jnp.float32),
            pltpu.VMEM((_BPW, _DIM), jnp.float32),
            pltpu.SemaphoreType.DMA,
            pltpu.SemaphoreType.DMA,
        ],
    )
    def k(x_hbm, table_hbm, out_hbm, idx_v, rows_v, out_v, sem_a, sem_b):
        wid = lax.axis_index("s") * 2 + lax.axis_index("c")
        base = wid * _BPW
        pltpu.sync_copy(x_hbm.at[pl.ds(base, _BPW)], idx_v)

        def issue(g, buf, sem):
            for j in range(_G):
                for h in range(2):
                    pltpu.async_copy(
                        table_hbm.at[idx_v.at[g * _G + j, h]],
                        rows_v.at[buf, pl.ds((2 * j + h) * _LP, _LP)],
                        sem)

        def drain(buf, sem):
            pltpu.make_async_copy(
                out_hbm.at[pl.ds(0, _GR)], rows_v.at[buf], sem).wait()

        def accumulate(g, buf):
            for j in range(_G):
                cb = 2 * j * _LP

                def acc_body(q, accs):
                    accs = list(accs)
                    base = cb + q * 16
                    for rr in range(16):
                        gidx = rr % 4
                        for c in range(4):
                            accs[4 * gidx + c] = (
                                accs[4 * gidx + c]
                                + rows_v[buf, base + rr, pl.ds(16 * c, 16)])
                    return tuple(accs)

                accs = lax.fori_loop(
                    0, 2 * _LP // 16, acc_body,
                    tuple(jnp.zeros((16,), jnp.float32) for _ in range(16)))
                for c in range(4):
                    tot = ((accs[c] + accs[4 + c])
                           + (accs[8 + c] + accs[12 + c]))
                    cs = pl.ds(16 * c, 16)
                    corr = ((rows_v[buf, cb + 0, cs] + rows_v[buf, cb + 1, cs])
                            + (rows_v[buf, cb + 2, cs] + rows_v[buf, cb + 3, cs]))
                    corr += ((rows_v[buf, cb + _LP + 0, cs]
                              + rows_v[buf, cb + _LP + 1, cs])
                             + (rows_v[buf, cb + _LP + 2, cs]
                                + rows_v[buf, cb + _LP + 3, cs]))
                    out_v[g * _G + j, cs] = (tot - corr) * (1.0 / _L)

        issue(0, 0, sem_a)

        def step(s, carry):
            g0 = 2 * s
            issue(g0 + 1, 1, sem_b)
            drain(0, sem_a)
            accumulate(g0, 0)

            @pl.when(s < _NG // 2 - 1)
            def _():
                issue(g0 + 2, 0, sem_a)

            drain(1, sem_b)
            accumulate(g0 + 1, 1)
            return carry

        lax.fori_loop(0, _NG // 2, step, 0)
        pltpu.sync_copy(out_v, out_hbm.at[pl.ds(base, _BPW)])

    return k


_kernel_call = _make_kernel()


def kernel(X, table):
    X2 = X.reshape(_B, 2, _L // 2)
    Xp = jnp.concatenate([X2, X2[:, :, : _LP - _L // 2]], axis=2)
    return _kernel_call(Xp, table)


# speedup vs baseline: 1.8795x; 1.0061x over previous
import functools

import jax
import jax.numpy as jnp
from jax import lax
from jax.experimental import pallas as pl
from jax.experimental.pallas import tpu as pltpu
from jax.experimental.pallas import tpu_sc as plsc

_B = 4096
_L = 200
_DIM = 64
_LP = 104
_OV = 2 * _LP - _L
_NW = 32
_BPW = _B // _NW
_G = 2
_NG = _BPW // _G
_GR = 2 * _LP * _G


def _make_kernel():
    mesh = plsc.VectorSubcoreMesh(core_axis_name="c", subcore_axis_name="s")

    @functools.partial(
        pl.kernel,
        mesh=mesh,
        out_type=jax.ShapeDtypeStruct((_B, _DIM), jnp.float32),
        compiler_params=pltpu.CompilerParams(use_tc_tiling_on_sc=False),
        scratch_types=[
            pltpu.VMEM((_BPW, _L), jnp.int32),
            pltpu.VMEM((2, _GR, _DIM), jnp.float32),
            pltpu.VMEM((_BPW, _DIM), jnp.float32),
            pltpu.SemaphoreType.DMA,
            pltpu.SemaphoreType.DMA,
        ],
    )
    def k(x_hbm, table_hbm, out_hbm, idx_v, rows_v, out_v, sem_a, sem_b):
        wid = lax.axis_index("s") * 2 + lax.axis_index("c")
        base = wid * _BPW
        pltpu.sync_copy(x_hbm.at[pl.ds(base, _BPW)], idx_v)

        def issue(g, buf, sem):
            for j in range(_G):
                b = g * _G + j
                for h, off in enumerate((0, _L - _LP)):
                    pltpu.async_copy(
                        table_hbm.at[idx_v.at[b, pl.ds(off, _LP)]],
                        rows_v.at[buf, pl.ds((2 * j + h) * _LP, _LP)],
                        sem)

        def drain(buf, sem):
            pltpu.make_async_copy(
                out_hbm.at[pl.ds(0, _GR)], rows_v.at[buf], sem).wait()

        def accumulate(g, buf):
            for j in range(_G):
                cb = 2 * j * _LP

                def acc_body(q, accs):
                    accs = list(accs)
                    rbase = cb + q * 16
                    for rr in range(16):
                        gidx = rr % 4
                        for c in range(4):
                            accs[4 * gidx + c] = (
                                accs[4 * gidx + c]
                                + rows_v[buf, rbase + rr, pl.ds(16 * c, 16)])
                    return tuple(accs)

                accs = lax.fori_loop(
                    0, 2 * _LP // 16, acc_body,
                    tuple(jnp.zeros((16,), jnp.float32) for _ in range(16)))
                for c in range(4):
                    tot = ((accs[c] + accs[4 + c])
                           + (accs[8 + c] + accs[12 + c]))
                    cs = pl.ds(16 * c, 16)
                    corr = ((rows_v[buf, cb + _LP + 0, cs]
                             + rows_v[buf, cb + _LP + 1, cs])
                            + (rows_v[buf, cb + _LP + 2, cs]
                               + rows_v[buf, cb + _LP + 3, cs]))
                    corr += ((rows_v[buf, cb + _LP + 4, cs]
                              + rows_v[buf, cb + _LP + 5, cs])
                             + (rows_v[buf, cb + _LP + 6, cs]
                                + rows_v[buf, cb + _LP + 7, cs]))
                    out_v[g * _G + j, cs] = (tot - corr) * (1.0 / _L)

        issue(0, 0, sem_a)

        def step(s, carry):
            g0 = 2 * s
            issue(g0 + 1, 1, sem_b)
            drain(0, sem_a)
            accumulate(g0, 0)

            @pl.when(s < _NG // 2 - 1)
            def _():
                issue(g0 + 2, 0, sem_a)

            drain(1, sem_b)
            accumulate(g0 + 1, 1)
            return carry

        lax.fori_loop(0, _NG // 2, step, 0)
        pltpu.sync_copy(out_v, out_hbm.at[pl.ds(base, _BPW)])

    return k


_kernel_call = _make_kernel()


def kernel(X, table):
    return _kernel_call(X, table)
